# Initial kernel scaffold; baseline (speedup 1.0000x reference)
import jax
import jax.numpy as jnp
from jax.experimental import pallas as pl


def kernel(coords, feats, kernel, stride):
    raise NotImplementedError("write your pallas kernel here")


# speedup vs baseline: 1.6548x; 1.6548x over previous
import functools

import jax
import jax.numpy as jnp
from jax import lax
from jax.experimental import pallas as pl
from jax.experimental.pallas import tpu as pltpu
from jax.experimental.pallas import tpu_sc as plsc

N = 100000
CIN = 32
COUT = 32
NH = 16
BEV0 = 256
BEV1 = 256
BATCH = 4
V = BATCH * BEV0 * BEV1

BN = 2048
NBLK = 49
NPAD = BN * NBLK

NC = 2
NS = 16
L = 16
WN = 128
NCHUNK = 8
CHUNK = V // NCHUNK
TRASH = 1024
CROWS = CHUNK + TRASH
ZROWS = CROWS // NS
ZB = 132
WBROWS = CHUNK // NS
PTS_PER_TILE = NPAD // NS
NWIN = PTS_PER_TILE // WN
NPASS = NCHUNK // NC


def _mm_body(h_ref, f_ref, w_ref, y_ref):
    h = h_ref[...]
    f = f_ref[...]
    ft = jnp.tile(f, (1, NH))
    col = lax.broadcasted_iota(jnp.int32, (BN, NH * CIN), 1) // CIN
    xe = jnp.where(col == h[:, None], ft, 0.0)
    y_ref[...] = jnp.dot(xe, w_ref[...], preferred_element_type=jnp.float32)


def _point_matmul(h_p, feats_p, wflat):
    return pl.pallas_call(
        _mm_body,
        grid=(NBLK,),
        in_specs=[
            pl.BlockSpec((BN,), lambda i: (i,)),
            pl.BlockSpec((BN, CIN), lambda i: (i, 0)),
            pl.BlockSpec((NH * CIN, COUT), lambda i: (0, 0)),
        ],
        out_specs=pl.BlockSpec((BN, COUT), lambda i: (i, 0)),
        out_shape=jax.ShapeDtypeStruct((NPAD, COUT), jnp.float32),
    )(h_p, feats_p, wflat)


def _scatter_body(y_hbm, idx_hbm, out_hbm, acc_sh, idxw, idx2, updw, zbuf):
    cid = lax.axis_index("c")
    sid = lax.axis_index("s")
    lane = lax.broadcasted_iota(jnp.int32, (L,), 0)

    zero16 = jnp.zeros((L,), jnp.float32)

    def zero_row(i, c):
        zbuf[i, pl.ds(0, L)] = zero16
        zbuf[i, pl.ds(L, L)] = zero16
        return c

    lax.fori_loop(0, ZB, zero_row, 0)

    for p in range(NPASS):
        base = (NC * p + cid) * CHUNK

        for z in range(ZROWS // ZB):
            pltpu.sync_copy(zbuf, acc_sh.at[pl.ds(sid * ZROWS + z * ZB, ZB)])
        plsc.subcore_barrier()

        def win_body(w, c):
            start = pl.multiple_of(sid * PTS_PER_TILE + w * WN, WN)
            pltpu.sync_copy(idx_hbm.at[pl.ds(start, WN)], idxw)
            pltpu.sync_copy(y_hbm.at[pl.ds(start, WN)], updw)
            for j in range(WN // L):
                v = idxw[pl.ds(j * L, L)]
                loc = v - base
                oob = (loc < 0) | (loc >= CHUNK)
                tr = CHUNK + ((lane + (j * L) + sid * WN) & (TRASH - 1))
                idx2[pl.ds(j * L, L)] = jnp.where(oob, tr, loc)
            pltpu.sync_copy(updw, acc_sh.at[idx2], add=True)
            return c

        lax.fori_loop(0, NWIN, win_body, 0)
        plsc.subcore_barrier()

        pltpu.sync_copy(
            acc_sh.at[pl.ds(sid * WBROWS, WBROWS)],
            out_hbm.at[pl.ds(base + sid * WBROWS, WBROWS)],
        )
        plsc.subcore_barrier()


def _scatter_add(y_p, idx_p):
    mesh = plsc.VectorSubcoreMesh(
        core_axis_name="c", subcore_axis_name="s", num_cores=NC, num_subcores=NS
    )
    run = pl.kernel(
        _scatter_body,
        out_type=jax.ShapeDtypeStruct((V, COUT), jnp.float32),
        mesh=mesh,
        compiler_params=pltpu.CompilerParams(use_tc_tiling_on_sc=False),
        scratch_types=[
            pltpu.VMEM_SHARED((CROWS, COUT), jnp.float32),
            pltpu.VMEM((WN,), jnp.int32),
            pltpu.VMEM((WN,), jnp.int32),
            pltpu.VMEM((WN, COUT), jnp.float32),
            pltpu.VMEM((ZB, COUT), jnp.float32),
        ],
    )
    return run(y_p, idx_p)


def kernel(coords, feats, kernel, stride):
    c = coords.astype(jnp.int32)
    h = c[:, 1] // stride
    idx = c[:, 3] * (BEV0 * BEV1) + (c[:, 0] // stride) * BEV1 + (c[:, 2] // stride)

    pad = NPAD - N
    h_p = jnp.pad(h, (0, pad))
    idx_p = jnp.pad(idx, (0, pad))
    feats_p = jnp.pad(feats, ((0, pad), (0, 0)))
    wflat = kernel.reshape(NH * CIN, COUT)

    y_p = _point_matmul(h_p, feats_p, wflat)
    table = _scatter_add(y_p, idx_p.astype(jnp.int32))
    out = table.reshape(BATCH, BEV0, BEV1, COUT)
    return jnp.transpose(out, (0, 3, 1, 2))
